# baseline (speedup 1.0000x reference)
import functools

import jax
import jax.numpy as jnp
from jax import lax
from jax.experimental import pallas as pl
from jax.experimental.pallas import tpu as pltpu
from jax.experimental.pallas import tpu_sc as plsc

_MAX_POS = 1024
_LANES = 16
_CH = 112
_GRP = 4


@functools.lru_cache(maxsize=None)
def _build(n, vocab, d):
    info = plsc.get_sparse_core_info()
    nw = info.num_cores * info.num_subcores
    nch = -(-n // (nw * _CH))
    bpw = nch * _CH
    assert n % 8 == 0 and bpw % 8 == 0 and bpw % _LANES == 0 and n >= bpw
    assert nch % _GRP == 0

    mesh = plsc.VectorSubcoreMesh(core_axis_name="c", subcore_axis_name="s")

    def body(table_hbm, idx_hbm, out_hbm, idx_v, buf_v, table_sh, gsem, wsem, ssem):
        sid = lax.axis_index("s")
        wid = sid * info.num_cores + lax.axis_index("c")
        base = jnp.minimum(wid * bpw, n - bpw)

        @pl.when(sid == 0)
        def _():
            pltpu.async_copy(table_hbm, table_sh, ssem)

        pltpu.sync_copy(idx_hbm.at[pl.ds(pl.multiple_of(base, 8), bpw)], idx_v)

        def clamp(i, carry):
            s = pl.ds(i * _LANES, _LANES)
            idx_v[s] = jnp.minimum(jnp.maximum(idx_v[s], 0), _MAX_POS)
            return carry

        lax.fori_loop(0, bpw // _LANES, clamp, 0)

        @pl.when(sid == 0)
        def _():
            pltpu.make_async_copy(table_hbm, table_sh, ssem).wait()

        plsc.subcore_barrier()

        nslot = nch // _GRP

        def fire_gather(ck, b, q):
            return pltpu.async_copy(
                table_sh.at[idx_v.at[pl.ds(ck * _CH, _CH)]],
                buf_v.at[b, pl.ds(q * _CH, _CH)],
                gsem.at[b],
            )

        def fire_write(s, b):
            off = pl.multiple_of(base + s * _GRP * _CH, 8)
            return pltpu.async_copy(
                buf_v.at[b],
                out_hbm.at[pl.ds(off, _GRP * _CH)],
                wsem.at[b],
            )

        writes = {}
        for s in range(nslot):
            b = s % 2
            if s >= 2:
                writes.pop(s - 2).wait()
            handles = [fire_gather(s * _GRP + q, b, q) for q in range(_GRP)]
            for h in handles:
                h.wait()
            writes[s] = fire_write(s, b)
        for j in sorted(writes):
            writes.pop(j).wait()

    return pl.kernel(
        body,
        mesh=mesh,
        out_type=jax.ShapeDtypeStruct((n, d), jnp.float32),
        scratch_types=[
            pltpu.VMEM((bpw,), jnp.int32),
            pltpu.VMEM((2, _GRP * _CH, d), jnp.float32),
            pltpu.VMEM_SHARED((vocab, d), jnp.float32),
            pltpu.SemaphoreType.DMA((2,)),
            pltpu.SemaphoreType.DMA((2,)),
            pltpu.SemaphoreType.DMA,
        ],
    )


def kernel(node_idx, pos_embedding_weight):
    n = node_idx.shape[0]
    vocab, d = pos_embedding_weight.shape
    f = _build(n, vocab, d)
    return f(pos_embedding_weight, node_idx.astype(jnp.int32))


# speedup vs baseline: 1.0229x; 1.0229x over previous
import functools

import jax
import jax.numpy as jnp
from jax import lax
from jax.experimental import pallas as pl
from jax.experimental.pallas import tpu as pltpu
from jax.experimental.pallas import tpu_sc as plsc

_MAX_POS = 1024
_LANES = 16
_CH = 112
_NBUF = 4
_LAG = 2


@functools.lru_cache(maxsize=None)
def _build(n, vocab, d):
    info = plsc.get_sparse_core_info()
    nw = info.num_cores * info.num_subcores
    nch = -(-n // (nw * _CH))
    bpw = nch * _CH
    assert n % 8 == 0 and bpw % 8 == 0 and _CH % _LANES == 0 and n >= bpw

    mesh = plsc.VectorSubcoreMesh(core_axis_name="c", subcore_axis_name="s")

    def body(table_hbm, idx_hbm, out_hbm, idx_v, buf_v, table_sh, gsem, wsem, ssem, isem):
        sid = lax.axis_index("s")
        wid = sid * info.num_cores + lax.axis_index("c")
        base = jnp.minimum(wid * bpw, n - bpw)

        @pl.when(sid == 0)
        def _():
            pltpu.async_copy(table_hbm, table_sh, ssem)

        idx_load = pltpu.async_copy(
            idx_hbm.at[pl.ds(pl.multiple_of(base, 8), bpw)], idx_v, isem
        )

        @pl.when(sid == 0)
        def _():
            pltpu.make_async_copy(table_hbm, table_sh, ssem).wait()

        plsc.subcore_barrier()
        idx_load.wait()

        def clamp(ck):
            for j in range(_CH // _LANES):
                s = pl.ds(ck * _CH + j * _LANES, _LANES)
                idx_v[s] = jnp.minimum(jnp.maximum(idx_v[s], 0), _MAX_POS)

        def fire_gather(ck):
            b = ck % _NBUF
            return pltpu.async_copy(
                table_sh.at[idx_v.at[pl.ds(ck * _CH, _CH)]],
                buf_v.at[b],
                gsem.at[b],
            )

        def fire_write(ck):
            b = ck % _NBUF
            off = pl.multiple_of(base + ck * _CH, 8)
            return pltpu.async_copy(
                buf_v.at[b],
                out_hbm.at[pl.ds(off, _CH)],
                wsem.at[b],
            )

        clamp(0)
        gathers = {}
        writes = {}
        for t in range(nch + _LAG):
            if t < nch:
                if t >= _NBUF:
                    writes.pop(t - _NBUF).wait()
                gathers[t] = fire_gather(t)
                if t + 1 < nch:
                    clamp(t + 1)
            j = t - _LAG
            if j >= 0:
                gathers.pop(j).wait()
                writes[j] = fire_write(j)
        for j in sorted(writes):
            writes.pop(j).wait()

    return pl.kernel(
        body,
        mesh=mesh,
        out_type=jax.ShapeDtypeStruct((n, d), jnp.float32),
        scratch_types=[
            pltpu.VMEM((bpw,), jnp.int32),
            pltpu.VMEM((_NBUF, _CH, d), jnp.float32),
            pltpu.VMEM_SHARED((vocab, d), jnp.float32),
            pltpu.SemaphoreType.DMA((_NBUF,)),
            pltpu.SemaphoreType.DMA((_NBUF,)),
            pltpu.SemaphoreType.DMA,
            pltpu.SemaphoreType.DMA,
        ],
    )


def kernel(node_idx, pos_embedding_weight):
    n = node_idx.shape[0]
    vocab, d = pos_embedding_weight.shape
    f = _build(n, vocab, d)
    return f(pos_embedding_weight, node_idx.astype(jnp.int32))


# speedup vs baseline: 1.0338x; 1.0107x over previous
import functools

import jax
import jax.numpy as jnp
from jax import lax
from jax.experimental import pallas as pl
from jax.experimental.pallas import tpu as pltpu
from jax.experimental.pallas import tpu_sc as plsc

_MAX_POS = 1024
_LANES = 16
_CH = 112
_NBUF = 4
_LAG = 2


@functools.lru_cache(maxsize=None)
def _build(n, vocab, d):
    info = plsc.get_sparse_core_info()
    nw = info.num_cores * info.num_subcores
    nch = -(-n // (nw * _CH))
    bpw = nch * _CH
    assert n % 8 == 0 and bpw % 8 == 0 and bpw % _LANES == 0 and n >= bpw

    mesh = plsc.VectorSubcoreMesh(core_axis_name="c", subcore_axis_name="s")

    def body(table_hbm, idx_hbm, out_hbm, idx_v, buf_v, table_sh, gsem, wsem, ssem):
        sid = lax.axis_index("s")
        wid = sid * info.num_cores + lax.axis_index("c")
        base = jnp.minimum(wid * bpw, n - bpw)

        @pl.when(sid == 0)
        def _():
            pltpu.async_copy(table_hbm, table_sh, ssem)

        pltpu.sync_copy(idx_hbm.at[pl.ds(pl.multiple_of(base, 8), bpw)], idx_v)

        def clamp(i, carry):
            s = pl.ds(i * _LANES, _LANES)
            idx_v[s] = jnp.minimum(jnp.maximum(idx_v[s], 0), _MAX_POS)
            return carry

        lax.fori_loop(0, bpw // _LANES, clamp, 0)

        @pl.when(sid == 0)
        def _():
            pltpu.make_async_copy(table_hbm, table_sh, ssem).wait()

        plsc.subcore_barrier()

        def fire_gather(ck):
            b = ck % _NBUF
            return pltpu.async_copy(
                table_sh.at[idx_v.at[pl.ds(ck * _CH, _CH)]],
                buf_v.at[b],
                gsem.at[b],
            )

        def fire_write(ck):
            b = ck % _NBUF
            off = pl.multiple_of(base + ck * _CH, 8)
            return pltpu.async_copy(
                buf_v.at[b],
                out_hbm.at[pl.ds(off, _CH)],
                wsem.at[b],
            )

        gathers = {}
        writes = {}
        for t in range(nch + _LAG):
            if t < nch:
                if t >= _NBUF:
                    writes.pop(t - _NBUF).wait()
                gathers[t] = fire_gather(t)
            j = t - _LAG
            if j >= 0:
                gathers.pop(j).wait()
                writes[j] = fire_write(j)
        for j in sorted(writes):
            writes.pop(j).wait()

    return pl.kernel(
        body,
        mesh=mesh,
        out_type=jax.ShapeDtypeStruct((n, d), jnp.float32),
        scratch_types=[
            pltpu.VMEM((bpw,), jnp.int32),
            pltpu.VMEM((_NBUF, _CH, d), jnp.float32),
            pltpu.VMEM_SHARED((vocab, d), jnp.float32),
            pltpu.SemaphoreType.DMA((_NBUF,)),
            pltpu.SemaphoreType.DMA((_NBUF,)),
            pltpu.SemaphoreType.DMA,
        ],
    )


def kernel(node_idx, pos_embedding_weight):
    n = node_idx.shape[0]
    vocab, d = pos_embedding_weight.shape
    f = _build(n, vocab, d)
    return f(pos_embedding_weight, node_idx.astype(jnp.int32))
